# baseline (speedup 1.0000x reference)
import functools

import jax
import jax.numpy as jnp
from jax import lax
from jax.experimental import pallas as pl
from jax.experimental.pallas import tpu as pltpu
from jax.experimental.pallas import tpu_sc as plsc

BATCH = 16384
EMBED_DIM = 32
TOTAL = 2 * BATCH
NUM_E = 1000000
PACK = 128 // EMBED_DIM
QROWS = NUM_E // PACK

_info = plsc.get_sparse_core_info()
NC, NS, L = _info.num_cores, _info.num_subcores, _info.num_lanes
NW = NC * NS
B_PER_W = TOTAL // NW
CHUNK = 128
NCHUNK = B_PER_W // CHUNK
GROUPS = CHUNK // L

NBLK = NUM_E // 128
BLK_PER_W = NBLK // NW
NPAIR = BLK_PER_W // 2
REM_FULL = NBLK - BLK_PER_W * NW
PART_E = NUM_E - NBLK * 128

_mesh = plsc.VectorSubcoreMesh(core_axis_name="c", subcore_axis_name="s")

_qshape = jax.ShapeDtypeStruct((QROWS, 128), jnp.float32)


@functools.partial(
    pl.kernel,
    mesh=_mesh,
    out_type=(_qshape, _qshape, _qshape, _qshape),
    compiler_params=pltpu.CompilerParams(
        needs_layout_passes=False, use_tc_tiling_on_sc=True
    ),
    scratch_types=[
        pltpu.VMEM((EMBED_DIM, 131), jnp.float32),
        pltpu.VMEM((EMBED_DIM, 131), jnp.float32),
        pltpu.VMEM((EMBED_DIM, 128), jnp.float32),
        pltpu.VMEM((EMBED_DIM, 128), jnp.float32),
        pltpu.SemaphoreType.DMA,
        pltpu.SemaphoreType.DMA,
        pltpu.SemaphoreType.DMA,
        pltpu.SemaphoreType.DMA,
    ],
)
def _convert_kernel(
    er_t, ei_t, rr_t, ri_t,
    er_x, ei_x, rr_x, ri_x,
    er_q, ei_q, rr_q, ri_q,
    src_a, src_b, quad_a, quad_b,
    sem_ia, sem_ib, sem_oa, sem_ob,
):
    wid = lax.axis_index("s") * NC + lax.axis_index("c")
    lane = lax.iota(jnp.int32, L)
    rows_lo = lane
    rows_hi = lane + L

    def shuffle(src_v, quad_v, nq):
        @plsc.parallel_loop(0, nq, unroll=8)
        def q_body(q):
            q4 = q * PACK
            for p in range(PACK):
                cols = jnp.full((L,), q4 + p, jnp.int32)
                lo = plsc.load_gather(src_v, [rows_lo, cols])
                quad_v[q, pl.ds(p * EMBED_DIM, L)] = lo
                hi = plsc.load_gather(src_v, [rows_hi, cols])
                quad_v[q, pl.ds(p * EMBED_DIM + L, L)] = hi

    def convert_table(tin, tout):
        base = wid * BLK_PER_W

        def fetch(blk, dst, sem):
            return pltpu.async_copy(
                tin.at[:, pl.ds(blk * 128, 128)], dst.at[:, pl.ds(0, 128)], sem
            )

        def drain_out(quad_v, sem):
            pltpu.make_async_copy(
                quad_v, tout.at[pl.ds(0, EMBED_DIM), :], sem
            ).wait()

        fetch(base, src_a, sem_ia)
        fetch(base + 1, src_b, sem_ib)

        def pair_body(i, c):
            blk_a = base + 2 * i
            blk_b = blk_a + 1

            @pl.when(i > 0)
            def _():
                drain_out(quad_a, sem_oa)
                drain_out(quad_b, sem_ob)

            pltpu.make_async_copy(
                tin.at[:, pl.ds(0, 128)], src_a.at[:, pl.ds(0, 128)], sem_ia
            ).wait()
            shuffle(src_a, quad_a, EMBED_DIM)
            pltpu.async_copy(
                quad_a, tout.at[pl.ds(blk_a * EMBED_DIM, EMBED_DIM), :], sem_oa
            )

            @pl.when(i < NPAIR - 1)
            def _():
                fetch(blk_a + 2, src_a, sem_ia)

            pltpu.make_async_copy(
                tin.at[:, pl.ds(0, 128)], src_b.at[:, pl.ds(0, 128)], sem_ib
            ).wait()
            shuffle(src_b, quad_b, EMBED_DIM)
            pltpu.async_copy(
                quad_b, tout.at[pl.ds(blk_b * EMBED_DIM, EMBED_DIM), :], sem_ob
            )

            @pl.when(i < NPAIR - 1)
            def _():
                fetch(blk_b + 2, src_b, sem_ib)

            return c

        lax.fori_loop(0, NPAIR, pair_body, 0)
        drain_out(quad_a, sem_oa)
        drain_out(quad_b, sem_ob)

    def convert_tail(tin, txq, tout, w):
        @pl.when(w < REM_FULL)
        def _():
            blk = NBLK - REM_FULL + w
            pltpu.sync_copy(
                tin.at[:, pl.ds(blk * 128, 128)], src_a.at[:, pl.ds(0, 128)]
            )
            shuffle(src_a, quad_a, EMBED_DIM)
            pltpu.sync_copy(
                quad_a, tout.at[pl.ds(blk * EMBED_DIM, EMBED_DIM), :]
            )

        @pl.when(w == REM_FULL)
        def _():
            nq = PART_E // PACK
            pltpu.sync_copy(txq, quad_a.at[pl.ds(0, nq), :])
            pltpu.sync_copy(
                quad_a.at[pl.ds(0, nq), :],
                tout.at[pl.ds(NBLK * EMBED_DIM, nq), :],
            )

    for tin, tout in ((er_t, er_q), (ei_t, ei_q), (rr_t, rr_q), (ri_t, ri_q)):
        convert_table(tin, tout)
    for tin, txq, tout in (
        (er_t, er_x, er_q),
        (ei_t, ei_x, ei_q),
        (rr_t, rr_x, rr_q),
        (ri_t, ri_x, ri_q),
    ):
        convert_tail(tin, txq, tout, wid)


@functools.partial(
    pl.kernel,
    mesh=_mesh,
    out_type=jax.ShapeDtypeStruct((TOTAL,), jnp.float32),
    compiler_params=pltpu.CompilerParams(
        needs_layout_passes=False, use_tc_tiling_on_sc=True
    ),
    scratch_types=[
        pltpu.VMEM((NCHUNK, CHUNK), jnp.int32),
        pltpu.VMEM((NCHUNK, CHUNK), jnp.int32),
        pltpu.VMEM((NCHUNK, CHUNK), jnp.int32),
        pltpu.VMEM((NCHUNK, CHUNK), jnp.int32),
        pltpu.VMEM((NCHUNK, CHUNK), jnp.int32),
        pltpu.VMEM((NCHUNK, CHUNK), jnp.int32),
        pltpu.VMEM((CHUNK, 128), jnp.float32),
        pltpu.VMEM((CHUNK, 128), jnp.float32),
        pltpu.VMEM((CHUNK, 128), jnp.float32),
        pltpu.VMEM((CHUNK, 128), jnp.float32),
        pltpu.VMEM((CHUNK, 128), jnp.float32),
        pltpu.VMEM((CHUNK, 128), jnp.float32),
        pltpu.VMEM((B_PER_W,), jnp.float32),
        pltpu.SemaphoreType.DMA,
    ],
)
def _score_kernel(
    s_hbm, r_hbm, o_hbm, so_hbm, ro_hbm, oo_hbm,
    er_q, ei_q, rr_q, ri_q,
    out_hbm,
    s_v, r_v, o_v, so_v, ro_v, oo_v,
    sr_v, si_v, or_v, oi_v, rr_v, ri_v,
    scores_v, sem,
):
    wid = lax.axis_index("s") * NC + lax.axis_index("c")

    pltpu.sync_copy(s_hbm.at[wid], s_v)
    pltpu.sync_copy(r_hbm.at[wid], r_v)
    pltpu.sync_copy(o_hbm.at[wid], o_v)
    pltpu.sync_copy(so_hbm.at[wid], so_v)
    pltpu.sync_copy(ro_hbm.at[wid], ro_v)
    pltpu.sync_copy(oo_hbm.at[wid], oo_v)

    lane = lax.iota(jnp.int32, L)

    def chunk_body(g, carry):
        cps = [
            pltpu.async_copy(er_q.at[s_v.at[g]], sr_v, sem),
            pltpu.async_copy(ei_q.at[s_v.at[g]], si_v, sem),
            pltpu.async_copy(er_q.at[o_v.at[g]], or_v, sem),
            pltpu.async_copy(ei_q.at[o_v.at[g]], oi_v, sem),
            pltpu.async_copy(rr_q.at[r_v.at[g]], rr_v, sem),
            pltpu.async_copy(ri_q.at[r_v.at[g]], ri_v, sem),
        ]
        for cp in cps:
            cp.wait()

        @plsc.parallel_loop(0, GROUPS, unroll=2)
        def group_body(g2):
            rows = g2 * L + lane
            offs = so_v[g, pl.ds(g2 * L, L)]
            offr = ro_v[g, pl.ds(g2 * L, L)]
            offo = oo_v[g, pl.ds(g2 * L, L)]
            acc = jnp.zeros((L,), jnp.float32)
            for d in range(EMBED_DIM):
                cs = offs + d
                cr = offr + d
                co = offo + d
                sr = plsc.load_gather(sr_v, [rows, cs])
                si = plsc.load_gather(si_v, [rows, cs])
                orr = plsc.load_gather(or_v, [rows, co])
                oii = plsc.load_gather(oi_v, [rows, co])
                rr = plsc.load_gather(rr_v, [rows, cr])
                ri = plsc.load_gather(ri_v, [rows, cr])
                acc = acc + (sr * (orr + oii) + si * (oii - orr) + (rr + ri))
            scores_v[pl.ds(g * CHUNK + g2 * L, L)] = acc

        return carry

    lax.fori_loop(0, NCHUNK, chunk_body, 0)

    pltpu.sync_copy(scores_v, out_hbm.at[pl.ds(wid * B_PER_W, B_PER_W)])


def kernel(positive, negative, ent_real, ent_imag, rel_real, rel_imag):
    def _tail(t):
        return t[NBLK * 128 :].reshape(PART_E // PACK, 128)

    er_q, ei_q, rr_q, ri_q = _convert_kernel(
        ent_real.T, ent_imag.T, rel_real.T, rel_imag.T,
        _tail(ent_real), _tail(ent_imag), _tail(rel_real), _tail(rel_imag),
    )
    trip = jnp.concatenate([positive, negative], axis=0)
    rows = (trip // PACK).astype(jnp.int32)
    offs = ((trip % PACK) * EMBED_DIM).astype(jnp.int32)
    s_idx = rows[:, 0].reshape(NW, NCHUNK, CHUNK)
    r_idx = rows[:, 1].reshape(NW, NCHUNK, CHUNK)
    o_idx = rows[:, 2].reshape(NW, NCHUNK, CHUNK)
    s_off = offs[:, 0].reshape(NW, NCHUNK, CHUNK)
    r_off = offs[:, 1].reshape(NW, NCHUNK, CHUNK)
    o_off = offs[:, 2].reshape(NW, NCHUNK, CHUNK)
    out = _score_kernel(
        s_idx, r_idx, o_idx, s_off, r_off, o_off, er_q, ei_q, rr_q, ri_q
    )
    return out[:BATCH], out[BATCH:]


# speedup vs baseline: 1.1082x; 1.1082x over previous
import functools

import jax
import jax.numpy as jnp
from jax import lax
from jax.experimental import pallas as pl
from jax.experimental.pallas import tpu as pltpu
from jax.experimental.pallas import tpu_sc as plsc

BATCH = 16384
EMBED_DIM = 32
TOTAL = 2 * BATCH

_info = plsc.get_sparse_core_info()
NC, NS, L = _info.num_cores, _info.num_subcores, _info.num_lanes
NW = NC * NS
B_PER_W = TOTAL // NW
CHUNK = 128
NCHUNK = B_PER_W // CHUNK

_mesh = plsc.VectorSubcoreMesh(core_axis_name="c", subcore_axis_name="s")


@functools.partial(
    pl.kernel,
    mesh=_mesh,
    out_type=jax.ShapeDtypeStruct((TOTAL,), jnp.float32),
    compiler_params=pltpu.CompilerParams(
        needs_layout_passes=False, use_tc_tiling_on_sc=False
    ),
    scratch_types=[
        pltpu.VMEM((NCHUNK, CHUNK), jnp.int32),
        pltpu.VMEM((NCHUNK, CHUNK), jnp.int32),
        pltpu.VMEM((NCHUNK, CHUNK), jnp.int32),
        pltpu.VMEM((CHUNK, EMBED_DIM), jnp.float32),
        pltpu.VMEM((CHUNK, EMBED_DIM), jnp.float32),
        pltpu.VMEM((CHUNK, EMBED_DIM), jnp.float32),
        pltpu.VMEM((CHUNK, EMBED_DIM), jnp.float32),
        pltpu.VMEM((CHUNK, EMBED_DIM), jnp.float32),
        pltpu.VMEM((CHUNK, EMBED_DIM), jnp.float32),
        pltpu.VMEM((B_PER_W,), jnp.float32),
        pltpu.VMEM((B_PER_W,), jnp.float32),
        pltpu.SemaphoreType.DMA,
    ],
)
def _complex_score_kernel(
    s_hbm, r_hbm, o_hbm,
    ent_real, ent_imag, rel_real, rel_imag,
    out_hbm,
    s_v, r_v, o_v,
    sr_v, si_v, or_v, oi_v, rr_v, ri_v,
    scores_v, acc_v, sem,
):
    wid = lax.axis_index("s") * NC + lax.axis_index("c")

    pltpu.sync_copy(s_hbm.at[wid], s_v)
    pltpu.sync_copy(r_hbm.at[wid], r_v)
    pltpu.sync_copy(o_hbm.at[wid], o_v)

    def chunk_body(g, carry):
        cps = [
            pltpu.async_copy(ent_real.at[s_v.at[g]], sr_v, sem),
            pltpu.async_copy(ent_imag.at[s_v.at[g]], si_v, sem),
            pltpu.async_copy(ent_real.at[o_v.at[g]], or_v, sem),
            pltpu.async_copy(ent_imag.at[o_v.at[g]], oi_v, sem),
            pltpu.async_copy(rel_real.at[r_v.at[g]], rr_v, sem),
            pltpu.async_copy(rel_imag.at[r_v.at[g]], ri_v, sem),
        ]
        for cp in cps:
            cp.wait()

        def zero_body(z, carry2):
            acc_v[pl.ds(g * CHUNK + z * L, L)] = jnp.zeros((L,), jnp.float32)
            return carry2

        lax.fori_loop(0, CHUNK // L, zero_body, 0)

        def trip_body(t, carry2):
            h = EMBED_DIM // 2
            sr0 = sr_v[t, pl.ds(0, h)]
            sr1 = sr_v[t, pl.ds(h, h)]
            si0 = si_v[t, pl.ds(0, h)]
            si1 = si_v[t, pl.ds(h, h)]
            or0 = or_v[t, pl.ds(0, h)]
            or1 = or_v[t, pl.ds(h, h)]
            oi0 = oi_v[t, pl.ds(0, h)]
            oi1 = oi_v[t, pl.ds(h, h)]
            rr0 = rr_v[t, pl.ds(0, h)]
            rr1 = rr_v[t, pl.ds(h, h)]
            ri0 = ri_v[t, pl.ds(0, h)]
            ri1 = ri_v[t, pl.ds(h, h)]
            v0 = sr0 * (or0 + oi0) + si0 * (oi0 - or0) + (rr0 + ri0)
            v1 = sr1 * (or1 + oi1) + si1 * (oi1 - or1) + (rr1 + ri1)
            v = v0 + v1
            slot = jnp.full((L,), g * CHUNK + t, jnp.int32)
            plsc.addupdate_scatter(acc_v, [slot], v)
            return carry2

        lax.fori_loop(0, CHUNK, trip_body, 0)
        return carry

    lax.fori_loop(0, NCHUNK, chunk_body, 0)

    def copy_body(z, carry):
        scores_v[pl.ds(z * L, L)] = acc_v[pl.ds(z * L, L)]
        return carry

    lax.fori_loop(0, B_PER_W // L, copy_body, 0)

    pltpu.sync_copy(scores_v, out_hbm.at[pl.ds(wid * B_PER_W, B_PER_W)])


def kernel(positive, negative, ent_real, ent_imag, rel_real, rel_imag):
    trip = jnp.concatenate([positive, negative], axis=0)
    s_idx = trip[:, 0].reshape(NW, NCHUNK, CHUNK)
    r_idx = trip[:, 1].reshape(NW, NCHUNK, CHUNK)
    o_idx = trip[:, 2].reshape(NW, NCHUNK, CHUNK)
    out = _complex_score_kernel(
        s_idx, r_idx, o_idx, ent_real, ent_imag, rel_real, rel_imag
    )
    return out[:BATCH], out[BATCH:]
